# baseline (speedup 1.0000x reference)
import functools

import jax
import jax.numpy as jnp
from jax import lax
from jax.experimental import pallas as pl
from jax.experimental.pallas import tpu as pltpu
from jax.experimental.pallas import tpu_sc as plsc

EMBED = 128
LANES = 16
NVEC = EMBED // LANES
NWORK = 32
GA = 104


def _sc_lookahead(t0, t1, t2, eos, idx_all, n_seq, seq_len):
    gb = seq_len - GA
    il = 3 * seq_len
    K = n_seq // NWORK
    mesh = plsc.VectorSubcoreMesh(core_axis_name="c", subcore_axis_name="s")

    @functools.partial(
        pl.kernel,
        out_type=jax.ShapeDtypeStruct((n_seq, seq_len, EMBED), jnp.float32),
        mesh=mesh,
        scratch_types=[
            pltpu.VMEM((2 * il,), jnp.int32),
            pltpu.VMEM((seq_len + 8, EMBED), jnp.float32),
            pltpu.VMEM((seq_len + 8, EMBED), jnp.float32),
            pltpu.VMEM((seq_len + 8, EMBED), jnp.float32),
            pltpu.VMEM((GA, EMBED), jnp.float32),
            pltpu.VMEM((seq_len - GA, EMBED), jnp.float32),
            pltpu.SemaphoreType.DMA,
            pltpu.SemaphoreType.DMA,
            pltpu.SemaphoreType.DMA,
            pltpu.SemaphoreType.DMA,
            pltpu.SemaphoreType.DMA,
        ],
    )
    def body(t0_h, t1_h, t2_h, eos_h, idx_h, out_h,
             idx_v, r0, r1, r2, obufa, obufb, gsa, gsb, isem, wsa, wsb):
        wid = lax.axis_index("s") * 2 + lax.axis_index("c")
        tables = (t0_h, t1_h, t2_h)
        slot = (r0, r1, r2)

        pltpu.sync_copy(eos_h, r0.at[seq_len])
        zeros = jnp.zeros((LANES,), jnp.float32)
        for v in range(NVEC):
            sl = pl.ds(v * LANES, LANES)
            r1[seq_len, sl] = zeros
            r2[seq_len, sl] = zeros

        def idx_off(i):
            return pl.multiple_of((i % 2) * il, 8)

        def fire_idx(i):
            pltpu.async_copy(
                idx_h.at[pl.ds((wid + NWORK * i) * il, il)],
                idx_v.at[pl.ds(idx_off(i), il)], isem)

        def wait_idx(i):
            pltpu.make_async_copy(
                idx_h.at[pl.ds(0, il)],
                idx_v.at[pl.ds(idx_off(i), il)], isem).wait()

        def fire_ga(i):
            base = idx_off(i)
            for t in range(3):
                pltpu.async_copy(
                    tables[t].at[idx_v.at[pl.ds(base + t * seq_len, GA)]],
                    slot[t].at[pl.ds(0, GA)], gsa)

        def fire_gb(i):
            base = idx_off(i)
            for t in range(3):
                pltpu.async_copy(
                    tables[t].at[idx_v.at[pl.ds(base + t * seq_len + GA, gb)]],
                    slot[t].at[pl.ds(GA, gb)], gsb)

        def wait_g(ofs, gn, sem):
            for t in range(3):
                pltpu.make_async_copy(
                    tables[t].at[pl.ds(0, gn)],
                    slot[t].at[pl.ds(ofs, gn)], sem).wait()

        def make_comp(obuf, ofs):
            def comp(s, carry):
                news = []
                for v in range(NVEC):
                    sl = pl.ds(v * LANES, LANES)
                    nxt = r0[s + 1, sl] + r1[s + 1, sl] + r2[s + 1, sl]
                    obuf[s - ofs, sl] = carry[v] + nxt
                    news.append(nxt)
                return tuple(news)
            return comp

        comp_a = make_comp(obufa, 0)
        comp_b = make_comp(obufb, GA)

        fire_idx(0)
        fire_idx(1)
        wait_idx(0)
        fire_ga(0)
        fire_gb(0)

        def item(i, _):
            n = wid + NWORK * i

            @pl.when(i > 0)
            def _():
                pltpu.make_async_copy(
                    obufa, out_h.at[n - NWORK, pl.ds(0, GA)], wsa).wait()

            wait_g(0, GA, gsa)
            init = tuple(
                r0[0, pl.ds(v * LANES, LANES)]
                + r1[0, pl.ds(v * LANES, LANES)]
                + r2[0, pl.ds(v * LANES, LANES)]
                for v in range(NVEC)
            )
            carry = lax.fori_loop(0, GA - 1, comp_a, init)

            @pl.when(i < K - 1)
            def _():
                wait_idx(i + 1)
                fire_ga(i + 1)

            wait_g(GA, gb, gsb)

            @pl.when(i < K - 2)
            def _():
                fire_idx(i + 2)

            news = []
            for v in range(NVEC):
                sl = pl.ds(v * LANES, LANES)
                nxt = r0[GA, sl] + r1[GA, sl] + r2[GA, sl]
                obufa[GA - 1, sl] = carry[v] + nxt
                news.append(nxt)
            pltpu.async_copy(obufa, out_h.at[n, pl.ds(0, GA)], wsa)

            @pl.when(i > 0)
            def _():
                pltpu.make_async_copy(
                    obufb, out_h.at[n - NWORK, pl.ds(GA, gb)], wsb).wait()

            lax.fori_loop(GA, seq_len, comp_b, tuple(news))
            pltpu.async_copy(obufb, out_h.at[n, pl.ds(GA, gb)], wsb)

            @pl.when(i < K - 1)
            def _():
                fire_gb(i + 1)

            return 0

        lax.fori_loop(0, K, item, 0)
        n_last = wid + NWORK * (K - 1)
        pltpu.make_async_copy(obufa, out_h.at[n_last, pl.ds(0, GA)], wsa).wait()
        pltpu.make_async_copy(obufb, out_h.at[n_last, pl.ds(GA, gb)], wsb).wait()

    return body(t0, t1, t2, eos, idx_all)


def kernel(table0, table1, table2, eos, position):
    n_seq, seq_len, _ = position.shape
    idx_all = position.transpose(0, 2, 1).reshape(-1)
    return _sc_lookahead(table0, table1, table2, eos, idx_all, n_seq, seq_len)
